# Initial kernel scaffold; baseline (speedup 1.0000x reference)
import jax
import jax.numpy as jnp
from jax.experimental import pallas as pl


def kernel(sem_seg, ctr_hmp, offsets):
    raise NotImplementedError("write your pallas kernel here")


# speedup vs baseline: 5.3959x; 5.3959x over previous
import functools

import jax
import jax.numpy as jnp
from jax.experimental import pallas as pl
from jax.experimental.pallas import tpu as pltpu

H = 384
W = 384
HW = H * W
THING_LIST = [1, 2]
THRESHOLD = 0.1
NMS_KERNEL = 7
K_MAX = 4096

ROWS = 16
UNROLL = 8


def _group_body(cy_ref, cx_ref, offy_ref, offx_ref, sem_ref, pan_ref, dist_ref):
    r = pl.program_id(0)
    row0 = (r * ROWS).astype(jnp.float32)
    iota_r = jax.lax.broadcasted_iota(jnp.int32, (ROWS, W), 0).astype(jnp.float32)
    iota_c = jax.lax.broadcasted_iota(jnp.int32, (ROWS, W), 1).astype(jnp.float32)
    ly = (row0 + iota_r) + offy_ref[...]
    lx = iota_c + offx_ref[...]

    def body(i, carry):
        bd, bi = carry
        c0 = i * UNROLL
        for u in range(UNROLL):
            c = c0 + u
            cy = cy_ref[c]
            cx = cx_ref[c]
            dy = ly - cy
            dx = lx - cx
            d = jnp.sqrt(dy * dy + dx * dx)
            m = d < bd
            bd = jnp.where(m, d, bd)
            bi = jnp.where(m, c + 1, bi)
        return bd, bi

    bd0 = jnp.full((ROWS, W), 1e5, jnp.float32)
    bi0 = jnp.zeros((ROWS, W), jnp.int32)
    bd, bi = jax.lax.fori_loop(0, K_MAX // UNROLL, body, (bd0, bi0))

    sem = sem_ref[...]
    thing = (sem == THING_LIST[0]) | (sem == THING_LIST[1])
    pan_ref[...] = jnp.where(thing, bi, 0)
    dist_ref[...] = bd


@jax.jit
def kernel(sem_seg, ctr_hmp, offsets):
    hmp = jnp.where(ctr_hmp > THRESHOLD, ctr_hmp, -1.0)
    pooled = jax.lax.reduce_window(hmp, -jnp.inf, jax.lax.max,
                                   (1, 1, NMS_KERNEL, NMS_KERNEL),
                                   (1, 1, 1, 1), 'SAME')
    keep = jnp.logical_and(hmp == pooled, hmp > 0.0)[0, 0]

    (idx,) = jnp.nonzero(keep.reshape(-1), size=K_MAX, fill_value=HW)
    valid = idx < HW
    idx_c = jnp.minimum(idx, HW - 1)
    cy = (idx_c // W).astype(jnp.float32)
    cx = (idx_c % W).astype(jnp.float32)
    ctr = jnp.stack([cy, cx], axis=-1)
    ctr = jnp.where(valid[:, None], ctr, 1e6)

    offy = offsets[0, 0]
    offx = offsets[0, 1]
    sem = sem_seg[0, 0]

    grid = (H // ROWS,)
    pan, dist = pl.pallas_call(
        _group_body,
        grid=grid,
        in_specs=[
            pl.BlockSpec(memory_space=pltpu.SMEM),
            pl.BlockSpec(memory_space=pltpu.SMEM),
            pl.BlockSpec((ROWS, W), lambda r: (r, 0)),
            pl.BlockSpec((ROWS, W), lambda r: (r, 0)),
            pl.BlockSpec((ROWS, W), lambda r: (r, 0)),
        ],
        out_specs=[
            pl.BlockSpec((ROWS, W), lambda r: (r, 0)),
            pl.BlockSpec((ROWS, W), lambda r: (r, 0)),
        ],
        out_shape=[
            jax.ShapeDtypeStruct((H, W), jnp.int32),
            jax.ShapeDtypeStruct((H, W), jnp.float32),
        ],
    )(ctr[:, 0], ctr[:, 1], offy, offx, sem)

    return pan[None], ctr[None], dist[None]


# speedup vs baseline: 9.0837x; 1.6835x over previous
import functools

import jax
import jax.numpy as jnp
from jax.experimental import pallas as pl
from jax.experimental.pallas import tpu as pltpu

H = 384
W = 384
HW = H * W
THING_LIST = [1, 2]
THRESHOLD = 0.1
NMS_KERNEL = 7
K_MAX = 4096

ROWS = 16
UNROLL = 8


def _group_body(cy_ref, cx_ref, offy_ref, offx_ref, sem_ref, pan_ref, dist_ref):
    r = pl.program_id(0)
    row0 = (r * ROWS).astype(jnp.float32)
    iota_r = jax.lax.broadcasted_iota(jnp.int32, (ROWS, W), 0).astype(jnp.float32)
    iota_c = jax.lax.broadcasted_iota(jnp.int32, (ROWS, W), 1).astype(jnp.float32)
    ly = (row0 + iota_r) + offy_ref[...]
    lx = iota_c + offx_ref[...]

    def body(i, carry):
        b2, bi = carry
        c0 = i * UNROLL
        for u in range(UNROLL):
            c = c0 + u
            cy = cy_ref[c]
            cx = cx_ref[c]
            dy = ly - cy
            dx = lx - cx
            d2 = dy * dy + dx * dx
            m = d2 < b2
            b2 = jnp.where(m, d2, b2)
            bi = jnp.where(m, c + 1, bi)
        return b2, bi

    b20 = jnp.full((ROWS, W), 1e10, jnp.float32)
    bi0 = jnp.zeros((ROWS, W), jnp.int32)
    b2, bi = jax.lax.fori_loop(0, K_MAX // UNROLL, body, (b20, bi0))

    sem = sem_ref[...]
    thing = (sem == THING_LIST[0]) | (sem == THING_LIST[1])
    pan_ref[...] = jnp.where(thing, bi, 0)
    dist_ref[...] = jnp.where(bi == 0, 1e5, jnp.sqrt(b2))


@jax.jit
def kernel(sem_seg, ctr_hmp, offsets):
    hmp = jnp.where(ctr_hmp > THRESHOLD, ctr_hmp, -1.0)
    pooled = jax.lax.reduce_window(hmp, -jnp.inf, jax.lax.max,
                                   (1, 1, NMS_KERNEL, NMS_KERNEL),
                                   (1, 1, 1, 1), 'SAME')
    keep = jnp.logical_and(hmp == pooled, hmp > 0.0)[0, 0]

    (idx,) = jnp.nonzero(keep.reshape(-1), size=K_MAX, fill_value=HW)
    valid = idx < HW
    idx_c = jnp.minimum(idx, HW - 1)
    cy = (idx_c // W).astype(jnp.float32)
    cx = (idx_c % W).astype(jnp.float32)
    ctr = jnp.stack([cy, cx], axis=-1)
    ctr = jnp.where(valid[:, None], ctr, 1e6)

    offy = offsets[0, 0]
    offx = offsets[0, 1]
    sem = sem_seg[0, 0]

    grid = (H // ROWS,)
    pan, dist = pl.pallas_call(
        _group_body,
        grid=grid,
        in_specs=[
            pl.BlockSpec(memory_space=pltpu.SMEM),
            pl.BlockSpec(memory_space=pltpu.SMEM),
            pl.BlockSpec((ROWS, W), lambda r: (r, 0)),
            pl.BlockSpec((ROWS, W), lambda r: (r, 0)),
            pl.BlockSpec((ROWS, W), lambda r: (r, 0)),
        ],
        out_specs=[
            pl.BlockSpec((ROWS, W), lambda r: (r, 0)),
            pl.BlockSpec((ROWS, W), lambda r: (r, 0)),
        ],
        out_shape=[
            jax.ShapeDtypeStruct((H, W), jnp.int32),
            jax.ShapeDtypeStruct((H, W), jnp.float32),
        ],
    )(ctr[:, 0], ctr[:, 1], offy, offx, sem)

    return pan[None], ctr[None], dist[None]


# speedup vs baseline: 9.4897x; 1.0447x over previous
import functools

import jax
import jax.numpy as jnp
from jax.experimental import pallas as pl
from jax.experimental.pallas import tpu as pltpu

H = 384
W = 384
HW = H * W
THING_LIST = [1, 2]
THRESHOLD = 0.1
NMS_KERNEL = 7
K_MAX = 4096

ROWS = 32
UNROLL = 8


def _group_body(cy_ref, cx_ref, offy_ref, offx_ref, sem_ref, pan_ref, dist_ref):
    r = pl.program_id(0)
    row0 = (r * ROWS).astype(jnp.float32)
    iota_r = jax.lax.broadcasted_iota(jnp.int32, (ROWS, W), 0).astype(jnp.float32)
    iota_c = jax.lax.broadcasted_iota(jnp.int32, (ROWS, W), 1).astype(jnp.float32)
    ly = (row0 + iota_r) + offy_ref[...]
    lx = iota_c + offx_ref[...]

    def body(i, carry):
        b2, bi = carry
        c0 = i * UNROLL
        for u in range(UNROLL):
            c = c0 + u
            cy = cy_ref[c]
            cx = cx_ref[c]
            dy = ly - cy
            dx = lx - cx
            d2 = dy * dy + dx * dx
            m = d2 < b2
            bi = jnp.where(m, c + 1, bi)
            b2 = jnp.minimum(b2, d2)
        return b2, bi

    b20 = jnp.full((ROWS, W), 1e10, jnp.float32)
    bi0 = jnp.zeros((ROWS, W), jnp.int32)
    b2, bi = jax.lax.fori_loop(0, K_MAX // UNROLL, body, (b20, bi0))

    sem = sem_ref[...]
    thing = (sem == THING_LIST[0]) | (sem == THING_LIST[1])
    pan_ref[...] = jnp.where(thing, bi, 0)
    dist_ref[...] = jnp.where(bi == 0, 1e5, jnp.sqrt(b2))


@jax.jit
def kernel(sem_seg, ctr_hmp, offsets):
    hmp = jnp.where(ctr_hmp > THRESHOLD, ctr_hmp, -1.0)
    pooled = jax.lax.reduce_window(hmp, -jnp.inf, jax.lax.max,
                                   (1, 1, NMS_KERNEL, NMS_KERNEL),
                                   (1, 1, 1, 1), 'SAME')
    keep = jnp.logical_and(hmp == pooled, hmp > 0.0)[0, 0]

    (idx,) = jnp.nonzero(keep.reshape(-1), size=K_MAX, fill_value=HW)
    valid = idx < HW
    idx_c = jnp.minimum(idx, HW - 1)
    cy = (idx_c // W).astype(jnp.float32)
    cx = (idx_c % W).astype(jnp.float32)
    ctr = jnp.stack([cy, cx], axis=-1)
    ctr = jnp.where(valid[:, None], ctr, 1e6)

    offy = offsets[0, 0]
    offx = offsets[0, 1]
    sem = sem_seg[0, 0]

    grid = (H // ROWS,)
    pan, dist = pl.pallas_call(
        _group_body,
        grid=grid,
        in_specs=[
            pl.BlockSpec(memory_space=pltpu.SMEM),
            pl.BlockSpec(memory_space=pltpu.SMEM),
            pl.BlockSpec((ROWS, W), lambda r: (r, 0)),
            pl.BlockSpec((ROWS, W), lambda r: (r, 0)),
            pl.BlockSpec((ROWS, W), lambda r: (r, 0)),
        ],
        out_specs=[
            pl.BlockSpec((ROWS, W), lambda r: (r, 0)),
            pl.BlockSpec((ROWS, W), lambda r: (r, 0)),
        ],
        out_shape=[
            jax.ShapeDtypeStruct((H, W), jnp.int32),
            jax.ShapeDtypeStruct((H, W), jnp.float32),
        ],
    )(ctr[:, 0], ctr[:, 1], offy, offx, sem)

    return pan[None], ctr[None], dist[None]
